# baseline (speedup 1.0000x reference)
import functools

import jax
import jax.numpy as jnp
from jax import lax
from jax.experimental import pallas as pl
from jax.experimental.pallas import tpu as pltpu
from jax.experimental.pallas import tpu_sc as plsc

POOL_T = 16
N_SEEDS = 4096
HOP1 = N_SEEDS * POOL_T
TOTAL = HOP1 + N_SEEDS
N_NODES = 100000

NUM_CORES = 2
NUM_SUBCORES = 16
NUM_WORKERS = NUM_CORES * NUM_SUBCORES
LANES = 16
SEEDS_PER_W = N_SEEDS // NUM_WORKERS
GROUPS = SEEDS_PER_W // LANES

_MESH = plsc.VectorSubcoreMesh(core_axis_name="c", subcore_axis_name="s")


@functools.partial(
    pl.kernel,
    out_type=(
        jax.ShapeDtypeStruct((POOL_T + 1, N_SEEDS, POOL_T), jnp.int32),
        jax.ShapeDtypeStruct((TOTAL,), jnp.int32),
        jax.ShapeDtypeStruct((POOL_T + 1, N_SEEDS, POOL_T), jnp.float32),
        jax.ShapeDtypeStruct((N_SEEDS, POOL_T), jnp.float32),
    ),
    mesh=_MESH,
    compiler_params=pltpu.CompilerParams(use_tc_tiling_on_sc=False),
    scratch_types=[
        pltpu.VMEM((SEEDS_PER_W,), jnp.int32),
        pltpu.VMEM((POOL_T, SEEDS_PER_W), jnp.int32),
        pltpu.VMEM((POOL_T, SEEDS_PER_W), jnp.int32),
        pltpu.VMEM((SEEDS_PER_W, POOL_T), jnp.int32),
        pltpu.VMEM((SEEDS_PER_W, POOL_T), jnp.float32),
        pltpu.VMEM((POOL_T, SEEDS_PER_W, POOL_T), jnp.int32),
        pltpu.VMEM((POOL_T, SEEDS_PER_W, POOL_T), jnp.float32),
        pltpu.SemaphoreType.DMA,
    ],
)
def _sampler_sc(node_ids_hbm, neigh_hbm, neigh_flat_hbm, alpha_hbm,
                neigh_out, ids_out, alpha_out, alpha0_out,
                idx0_v, scaled_v, idx1_v, neigh0_v, alpha0_v,
                neigh1_v, alpha1_v, sem):
    wid = lax.axis_index("s") * NUM_CORES + lax.axis_index("c")
    base = wid * SEEDS_PER_W

    pltpu.sync_copy(node_ids_hbm.at[pl.ds(base, SEEDS_PER_W)], idx0_v)

    cp_n0 = pltpu.async_copy(neigh_hbm.at[idx0_v], neigh0_v, sem)
    cp_a0 = pltpu.async_copy(alpha_hbm.at[idx0_v], alpha0_v, sem)

    for g in range(GROUPS):
        v16 = idx0_v[pl.ds(g * LANES, LANES)] * POOL_T
        for t in range(POOL_T):
            scaled_v[t, pl.ds(g * LANES, LANES)] = v16 + t

    cps_t = [
        pltpu.async_copy(neigh_flat_hbm.at[scaled_v.at[t]], idx1_v.at[t], sem)
        for t in range(POOL_T)
    ]

    cp_n0.wait()
    cp_a0.wait()
    pltpu.sync_copy(neigh0_v, neigh_out.at[POOL_T, pl.ds(base, SEEDS_PER_W), :])
    pltpu.sync_copy(idx0_v, ids_out.at[pl.ds(HOP1 + base, SEEDS_PER_W)])
    pltpu.sync_copy(alpha0_v, alpha_out.at[POOL_T, pl.ds(base, SEEDS_PER_W), :])
    pltpu.sync_copy(alpha0_v, alpha0_out.at[pl.ds(base, SEEDS_PER_W), :])

    for cp in cps_t:
        cp.wait()

    cps1 = []
    for t in range(POOL_T):
        cps1.append(pltpu.async_copy(neigh_hbm.at[idx1_v.at[t]], neigh1_v.at[t], sem))
        cps1.append(pltpu.async_copy(alpha_hbm.at[idx1_v.at[t]], alpha1_v.at[t], sem))
    for cp in cps1:
        cp.wait()

    for t in range(POOL_T):
        pltpu.sync_copy(idx1_v.at[t], ids_out.at[pl.ds(t * N_SEEDS + base, SEEDS_PER_W)])
    pltpu.sync_copy(neigh1_v, neigh_out.at[pl.ds(0, POOL_T), pl.ds(base, SEEDS_PER_W), :])
    pltpu.sync_copy(alpha1_v, alpha_out.at[pl.ds(0, POOL_T), pl.ds(base, SEEDS_PER_W), :])


def kernel(node_ids, neigh_table, alpha_table):
    one = lax.optimization_barrier(jnp.int32(1))
    neigh_flat = neigh_table.reshape(-1) * one
    neigh3, ids, alpha3, alpha0 = _sampler_sc(
        node_ids, neigh_table, neigh_flat, alpha_table)
    stacks = jnp.concatenate(
        [ids[:, None], neigh3.reshape(TOTAL, POOL_T)], axis=1)
    return stacks, alpha0, alpha3.reshape(TOTAL, POOL_T)


# speedup vs baseline: 1.2136x; 1.2136x over previous
import functools

import jax
import jax.numpy as jnp
from jax import lax
from jax.experimental import pallas as pl
from jax.experimental.pallas import tpu as pltpu
from jax.experimental.pallas import tpu_sc as plsc

POOL_T = 16
N_SEEDS = 4096
HOP1 = N_SEEDS * POOL_T
TOTAL = HOP1 + N_SEEDS
N_NODES = 100000

NUM_CORES = 2
NUM_SUBCORES = 16
NUM_WORKERS = NUM_CORES * NUM_SUBCORES
LANES = 16
SEEDS_PER_W = N_SEEDS // NUM_WORKERS
GROUPS = SEEDS_PER_W // LANES

_MESH = plsc.VectorSubcoreMesh(core_axis_name="c", subcore_axis_name="s")


@functools.partial(
    pl.kernel,
    out_type=(
        jax.ShapeDtypeStruct((POOL_T + 1, N_SEEDS, POOL_T), jnp.int32),
        jax.ShapeDtypeStruct((TOTAL,), jnp.int32),
        jax.ShapeDtypeStruct((POOL_T + 1, N_SEEDS, POOL_T), jnp.float32),
        jax.ShapeDtypeStruct((N_SEEDS, POOL_T), jnp.float32),
    ),
    mesh=_MESH,
    compiler_params=pltpu.CompilerParams(use_tc_tiling_on_sc=False),
    scratch_types=[
        pltpu.VMEM((SEEDS_PER_W,), jnp.int32),
        pltpu.VMEM((POOL_T, SEEDS_PER_W), jnp.int32),
        pltpu.VMEM((SEEDS_PER_W, POOL_T), jnp.int32),
        pltpu.VMEM((SEEDS_PER_W, POOL_T), jnp.float32),
        pltpu.VMEM((POOL_T, SEEDS_PER_W, POOL_T), jnp.int32),
        pltpu.VMEM((POOL_T, SEEDS_PER_W, POOL_T), jnp.float32),
        pltpu.SemaphoreType.DMA,
    ],
)
def _sampler_sc(node_ids_hbm, neigh_hbm, alpha_hbm,
                neigh_out, ids_out, alpha_out, alpha0_out,
                idx0_v, idx1_v, neigh0_v, alpha0_v,
                neigh1_v, alpha1_v, sem):
    wid = lax.axis_index("s") * NUM_CORES + lax.axis_index("c")
    base = wid * SEEDS_PER_W

    pltpu.sync_copy(node_ids_hbm.at[pl.ds(base, SEEDS_PER_W)], idx0_v)

    cp_n0 = pltpu.async_copy(neigh_hbm.at[idx0_v], neigh0_v, sem)
    cp_a0 = pltpu.async_copy(alpha_hbm.at[idx0_v], alpha0_v, sem)

    cp_n0.wait()

    lanes = lax.iota(jnp.int32, LANES)
    stage_masks = [(lanes & (1 << s)) == 0 for s in range(4)]
    shl_perm = [(lanes - (1 << s)) & (LANES - 1) for s in range(4)]
    shr_perm = [(lanes + (1 << s)) & (LANES - 1) for s in range(4)]

    def _lane_perm(v, perm):
        return jnp.take_along_axis(v, perm, axis=0)

    for g in range(GROUPS):
        vecs = [neigh0_v[g * LANES + i, :] for i in range(LANES)]
        for s in range(4):
            step = 1 << s
            nxt = list(vecs)
            for i in range(LANES):
                if i & step:
                    continue
                p = i + step
                lo, hi = vecs[i], vecs[p]
                nxt[i] = jnp.where(stage_masks[s], lo, _lane_perm(hi, shl_perm[s]))
                nxt[p] = jnp.where(stage_masks[s], _lane_perm(lo, shr_perm[s]), hi)
            vecs = nxt
        for t in range(LANES):
            idx1_v[t, pl.ds(g * LANES, LANES)] = vecs[t]

    cps1 = []
    for t in range(POOL_T):
        cps1.append(pltpu.async_copy(neigh_hbm.at[idx1_v.at[t]], neigh1_v.at[t], sem))
        cps1.append(pltpu.async_copy(alpha_hbm.at[idx1_v.at[t]], alpha1_v.at[t], sem))

    cp_a0.wait()
    pltpu.sync_copy(neigh0_v, neigh_out.at[POOL_T, pl.ds(base, SEEDS_PER_W), :])
    pltpu.sync_copy(idx0_v, ids_out.at[pl.ds(HOP1 + base, SEEDS_PER_W)])
    pltpu.sync_copy(alpha0_v, alpha_out.at[POOL_T, pl.ds(base, SEEDS_PER_W), :])
    pltpu.sync_copy(alpha0_v, alpha0_out.at[pl.ds(base, SEEDS_PER_W), :])

    for cp in cps1:
        cp.wait()

    for t in range(POOL_T):
        pltpu.sync_copy(idx1_v.at[t], ids_out.at[pl.ds(t * N_SEEDS + base, SEEDS_PER_W)])
    pltpu.sync_copy(neigh1_v, neigh_out.at[pl.ds(0, POOL_T), pl.ds(base, SEEDS_PER_W), :])
    pltpu.sync_copy(alpha1_v, alpha_out.at[pl.ds(0, POOL_T), pl.ds(base, SEEDS_PER_W), :])


def kernel(node_ids, neigh_table, alpha_table):
    neigh3, ids, alpha3, alpha0 = _sampler_sc(
        node_ids, neigh_table, alpha_table)
    stacks = jnp.concatenate(
        [ids[:, None], neigh3.reshape(TOTAL, POOL_T)], axis=1)
    return stacks, alpha0, alpha3.reshape(TOTAL, POOL_T)


# speedup vs baseline: 1.2190x; 1.0045x over previous
import functools

import jax
import jax.numpy as jnp
from jax import lax
from jax.experimental import pallas as pl
from jax.experimental.pallas import tpu as pltpu
from jax.experimental.pallas import tpu_sc as plsc

POOL_T = 16
N_SEEDS = 4096
HOP1 = N_SEEDS * POOL_T
TOTAL = HOP1 + N_SEEDS
N_NODES = 100000

NUM_CORES = 2
NUM_SUBCORES = 16
NUM_WORKERS = NUM_CORES * NUM_SUBCORES
LANES = 16
SEEDS_PER_W = N_SEEDS // NUM_WORKERS
GROUPS = SEEDS_PER_W // LANES

_MESH = plsc.VectorSubcoreMesh(core_axis_name="c", subcore_axis_name="s")

_LANE_IOTA = None


def _transpose16(load_vec, store_vec):
    lanes = lax.iota(jnp.int32, LANES)
    masks = [(lanes & (1 << s)) == 0 for s in range(4)]
    shl = [(lanes - (1 << s)) & (LANES - 1) for s in range(4)]
    shr = [(lanes + (1 << s)) & (LANES - 1) for s in range(4)]
    vecs = [load_vec(i) for i in range(LANES)]
    for s in range(4):
        step = 1 << s
        nxt = list(vecs)
        for i in range(LANES):
            if i & step:
                continue
            p = i + step
            lo, hi = vecs[i], vecs[p]
            nxt[i] = jnp.where(masks[s], lo, jnp.take_along_axis(hi, shl[s], axis=0))
            nxt[p] = jnp.where(masks[s], jnp.take_along_axis(lo, shr[s], axis=0), hi)
        vecs = nxt
    for d in range(LANES):
        store_vec(d, vecs[d])


@functools.partial(
    pl.kernel,
    out_type=(
        jax.ShapeDtypeStruct((POOL_T + 1, TOTAL), jnp.int32),
        jax.ShapeDtypeStruct((POOL_T, TOTAL), jnp.float32),
        jax.ShapeDtypeStruct((POOL_T, N_SEEDS), jnp.float32),
    ),
    mesh=_MESH,
    compiler_params=pltpu.CompilerParams(use_tc_tiling_on_sc=False),
    scratch_types=[
        pltpu.VMEM((SEEDS_PER_W,), jnp.int32),
        pltpu.VMEM((POOL_T, SEEDS_PER_W), jnp.int32),
        pltpu.VMEM((SEEDS_PER_W, POOL_T), jnp.int32),
        pltpu.VMEM((SEEDS_PER_W, POOL_T), jnp.float32),
        pltpu.VMEM((POOL_T, SEEDS_PER_W), jnp.float32),
        pltpu.VMEM((POOL_T, SEEDS_PER_W, POOL_T), jnp.int32),
        pltpu.VMEM((POOL_T, SEEDS_PER_W, POOL_T), jnp.float32),
        pltpu.VMEM((POOL_T, SEEDS_PER_W), jnp.int32),
        pltpu.VMEM((POOL_T, SEEDS_PER_W), jnp.float32),
        pltpu.SemaphoreType.DMA,
    ],
)
def _sampler_sc(node_ids_hbm, neigh_hbm, alpha_hbm,
                stackT, alphaT, alpha0T,
                idx0_v, idx1_v, neigh0_v, alpha0_v, a0T_v,
                neigh1_v, alpha1_v, tbuf_i, tbuf_f, sem):
    wid = lax.axis_index("s") * NUM_CORES + lax.axis_index("c")
    base = wid * SEEDS_PER_W

    pltpu.sync_copy(node_ids_hbm.at[pl.ds(base, SEEDS_PER_W)], idx0_v)

    cp_n0 = pltpu.async_copy(neigh_hbm.at[idx0_v], neigh0_v, sem)
    cp_a0 = pltpu.async_copy(alpha_hbm.at[idx0_v], alpha0_v, sem)
    cp_n0.wait()

    for g in range(GROUPS):
        _transpose16(
            lambda i: neigh0_v[g * LANES + i, :],
            lambda d, v: idx1_v.__setitem__((d, pl.ds(g * LANES, LANES)), v),
        )

    cps1 = []
    for t in range(POOL_T):
        cps1.append(pltpu.async_copy(neigh_hbm.at[idx1_v.at[t]], neigh1_v.at[t], sem))
        cps1.append(pltpu.async_copy(alpha_hbm.at[idx1_v.at[t]], alpha1_v.at[t], sem))

    cp_a0.wait()
    for g in range(GROUPS):
        _transpose16(
            lambda i: alpha0_v[g * LANES + i, :],
            lambda d, v: a0T_v.__setitem__((d, pl.ds(g * LANES, LANES)), v),
        )

    pltpu.sync_copy(idx0_v, stackT.at[0, pl.ds(HOP1 + base, SEEDS_PER_W)])
    pltpu.sync_copy(idx1_v, stackT.at[pl.ds(1, POOL_T), pl.ds(HOP1 + base, SEEDS_PER_W)])
    pltpu.sync_copy(a0T_v, alphaT.at[:, pl.ds(HOP1 + base, SEEDS_PER_W)])
    pltpu.sync_copy(a0T_v, alpha0T.at[:, pl.ds(base, SEEDS_PER_W)])

    for t in range(POOL_T):
        pltpu.sync_copy(idx1_v.at[t],
                        stackT.at[0, pl.ds(t * N_SEEDS + base, SEEDS_PER_W)])

    for cp in cps1:
        cp.wait()

    def _chunk_out(t, carry):
        col0 = pl.multiple_of(t * N_SEEDS + base, SEEDS_PER_W)
        for g in range(GROUPS):
            _transpose16(
                lambda i: neigh1_v[t, g * LANES + i, :],
                lambda d, v: tbuf_i.__setitem__((d, pl.ds(g * LANES, LANES)), v),
            )
            _transpose16(
                lambda i: alpha1_v[t, g * LANES + i, :],
                lambda d, v: tbuf_f.__setitem__((d, pl.ds(g * LANES, LANES)), v),
            )
        pltpu.sync_copy(tbuf_i, stackT.at[pl.ds(1, POOL_T), pl.ds(col0, SEEDS_PER_W)])
        pltpu.sync_copy(tbuf_f, alphaT.at[:, pl.ds(col0, SEEDS_PER_W)])
        return carry

    lax.fori_loop(0, POOL_T, _chunk_out, 0)


def kernel(node_ids, neigh_table, alpha_table):
    one_i = lax.optimization_barrier(jnp.int32(1))
    one_f = lax.optimization_barrier(jnp.float32(1))
    neigh_lin = (neigh_table.reshape(-1) * one_i).reshape(N_NODES, POOL_T)
    alpha_lin = (alpha_table.reshape(-1) * one_f).reshape(N_NODES, POOL_T)
    stackT, alphaT, alpha0T = _sampler_sc(node_ids, neigh_lin, alpha_lin)
    return stackT.T, alpha0T.T, alphaT.T


# speedup vs baseline: 1.8315x; 1.5024x over previous
import functools

import jax
import jax.numpy as jnp
from jax import lax
from jax.experimental import pallas as pl
from jax.experimental.pallas import tpu as pltpu
from jax.experimental.pallas import tpu_sc as plsc

POOL_T = 16
N_SEEDS = 4096
HOP1 = N_SEEDS * POOL_T
TOTAL = HOP1 + N_SEEDS
N_NODES = 100000

NUM_CORES = 2
NUM_SUBCORES = 16
NUM_WORKERS = NUM_CORES * NUM_SUBCORES
LANES = 16
SEEDS_PER_W = N_SEEDS // NUM_WORKERS
GROUPS = SEEDS_PER_W // LANES

_MESH = plsc.VectorSubcoreMesh(core_axis_name="c", subcore_axis_name="s")

_LANE_IOTA = None


def _transpose16(load_vec, store_vec):
    lanes = lax.iota(jnp.int32, LANES)
    masks = [(lanes & (1 << s)) == 0 for s in range(4)]
    shl = [(lanes - (1 << s)) & (LANES - 1) for s in range(4)]
    shr = [(lanes + (1 << s)) & (LANES - 1) for s in range(4)]
    vecs = [load_vec(i) for i in range(LANES)]
    for s in range(4):
        step = 1 << s
        nxt = list(vecs)
        for i in range(LANES):
            if i & step:
                continue
            p = i + step
            lo, hi = vecs[i], vecs[p]
            nxt[i] = jnp.where(masks[s], lo, jnp.take_along_axis(hi, shl[s], axis=0))
            nxt[p] = jnp.where(masks[s], jnp.take_along_axis(lo, shr[s], axis=0), hi)
        vecs = nxt
    for d in range(LANES):
        store_vec(d, vecs[d])


@functools.partial(
    pl.kernel,
    out_type=(
        jax.ShapeDtypeStruct((POOL_T + 1, TOTAL), jnp.int32),
        jax.ShapeDtypeStruct((POOL_T, TOTAL), jnp.float32),
        jax.ShapeDtypeStruct((POOL_T, N_SEEDS), jnp.float32),
    ),
    mesh=_MESH,
    compiler_params=pltpu.CompilerParams(use_tc_tiling_on_sc=False),
    scratch_types=[
        pltpu.VMEM((SEEDS_PER_W,), jnp.int32),
        pltpu.VMEM((POOL_T, SEEDS_PER_W), jnp.int32),
        pltpu.VMEM((SEEDS_PER_W, POOL_T), jnp.int32),
        pltpu.VMEM((SEEDS_PER_W, POOL_T), jnp.float32),
        pltpu.VMEM((POOL_T, SEEDS_PER_W), jnp.float32),
        pltpu.VMEM((POOL_T, SEEDS_PER_W, POOL_T), jnp.int32),
        pltpu.VMEM((POOL_T, SEEDS_PER_W, POOL_T), jnp.float32),
        pltpu.VMEM((POOL_T, SEEDS_PER_W), jnp.int32),
        pltpu.VMEM((POOL_T, SEEDS_PER_W), jnp.float32),
        pltpu.SemaphoreType.DMA,
    ],
)
def _sampler_sc(node_ids_hbm, neigh_hbm, alpha_hbm,
                stackT, alphaT, alpha0T,
                idx0_v, idx1_v, neigh0_v, alpha0_v, a0T_v,
                neigh1_v, alpha1_v, tbuf_i, tbuf_f, sem):
    wid = lax.axis_index("s") * NUM_CORES + lax.axis_index("c")
    base = wid * SEEDS_PER_W

    pltpu.sync_copy(node_ids_hbm.at[pl.ds(base, SEEDS_PER_W)], idx0_v)

    cp_n0 = pltpu.async_copy(neigh_hbm.at[idx0_v], neigh0_v, sem)
    cp_a0 = pltpu.async_copy(alpha_hbm.at[idx0_v], alpha0_v, sem)
    cp_n0.wait()

    for g in range(GROUPS):
        _transpose16(
            lambda i: neigh0_v[g * LANES + i, :],
            lambda d, v: idx1_v.__setitem__((d, pl.ds(g * LANES, LANES)), v),
        )

    cps1 = []
    for t in range(POOL_T):
        cps1.append(pltpu.async_copy(neigh_hbm.at[idx1_v.at[t]], neigh1_v.at[t], sem))
        cps1.append(pltpu.async_copy(alpha_hbm.at[idx1_v.at[t]], alpha1_v.at[t], sem))

    cp_a0.wait()
    for g in range(GROUPS):
        _transpose16(
            lambda i: alpha0_v[g * LANES + i, :],
            lambda d, v: a0T_v.__setitem__((d, pl.ds(g * LANES, LANES)), v),
        )

    pltpu.sync_copy(idx0_v, stackT.at[0, pl.ds(HOP1 + base, SEEDS_PER_W)])
    pltpu.sync_copy(idx1_v, stackT.at[pl.ds(1, POOL_T), pl.ds(HOP1 + base, SEEDS_PER_W)])
    pltpu.sync_copy(a0T_v, alphaT.at[:, pl.ds(HOP1 + base, SEEDS_PER_W)])
    pltpu.sync_copy(a0T_v, alpha0T.at[:, pl.ds(base, SEEDS_PER_W)])

    for t in range(POOL_T):
        pltpu.sync_copy(idx1_v.at[t],
                        stackT.at[0, pl.ds(t * N_SEEDS + base, SEEDS_PER_W)])

    for cp in cps1:
        cp.wait()

    def _chunk_out(t, carry):
        col0 = pl.multiple_of(t * N_SEEDS + base, SEEDS_PER_W)
        for g in range(GROUPS):
            _transpose16(
                lambda i: neigh1_v[t, g * LANES + i, :],
                lambda d, v: tbuf_i.__setitem__((d, pl.ds(g * LANES, LANES)), v),
            )
            _transpose16(
                lambda i: alpha1_v[t, g * LANES + i, :],
                lambda d, v: tbuf_f.__setitem__((d, pl.ds(g * LANES, LANES)), v),
            )
        pltpu.sync_copy(tbuf_i, stackT.at[pl.ds(1, POOL_T), pl.ds(col0, SEEDS_PER_W)])
        pltpu.sync_copy(tbuf_f, alphaT.at[:, pl.ds(col0, SEEDS_PER_W)])
        return carry

    lax.fori_loop(0, POOL_T, _chunk_out, 0)


def kernel(node_ids, neigh_table, alpha_table):
    stackT, alphaT, alpha0T = _sampler_sc(node_ids, neigh_table, alpha_table)
    return stackT.T, alpha0T.T, alphaT.T


# speedup vs baseline: 1.9445x; 1.0617x over previous
import functools

import jax
import jax.numpy as jnp
from jax import lax
from jax.experimental import pallas as pl
from jax.experimental.pallas import tpu as pltpu
from jax.experimental.pallas import tpu_sc as plsc

POOL_T = 16
N_SEEDS = 4096
HOP1 = N_SEEDS * POOL_T
TOTAL = HOP1 + N_SEEDS
N_NODES = 100000

NUM_CORES = 2
NUM_SUBCORES = 16
NUM_WORKERS = NUM_CORES * NUM_SUBCORES
LANES = 16
SEEDS_PER_W = N_SEEDS // NUM_WORKERS
GROUPS = SEEDS_PER_W // LANES

_MESH = plsc.VectorSubcoreMesh(core_axis_name="c", subcore_axis_name="s")


def _transpose16(load_vec, store_vec):
    lanes = lax.iota(jnp.int32, LANES)
    masks = [(lanes & (1 << s)) == 0 for s in range(4)]
    shl = [(lanes - (1 << s)) & (LANES - 1) for s in range(4)]
    shr = [(lanes + (1 << s)) & (LANES - 1) for s in range(4)]
    vecs = [load_vec(i) for i in range(LANES)]
    for s in range(4):
        step = 1 << s
        nxt = list(vecs)
        for i in range(LANES):
            if i & step:
                continue
            p = i + step
            lo, hi = vecs[i], vecs[p]
            nxt[i] = jnp.where(masks[s], lo, jnp.take_along_axis(hi, shl[s], axis=0))
            nxt[p] = jnp.where(masks[s], jnp.take_along_axis(lo, shr[s], axis=0), hi)
        vecs = nxt
    for d in range(LANES):
        store_vec(d, vecs[d])


@functools.partial(
    pl.kernel,
    out_type=jax.ShapeDtypeStruct((POOL_T + 1, TOTAL), jnp.int32),
    mesh=_MESH,
    compiler_params=pltpu.CompilerParams(use_tc_tiling_on_sc=False),
    scratch_types=[
        pltpu.VMEM((SEEDS_PER_W,), jnp.int32),
        pltpu.VMEM((POOL_T, SEEDS_PER_W), jnp.int32),
        pltpu.VMEM((SEEDS_PER_W, POOL_T), jnp.int32),
        pltpu.VMEM((POOL_T, SEEDS_PER_W, POOL_T), jnp.int32),
        pltpu.VMEM((POOL_T, SEEDS_PER_W), jnp.int32),
        pltpu.SemaphoreType.DMA,
    ],
)
def _neigh_sc(node_ids_hbm, neigh_hbm, stackT,
              idx0_v, idx1_v, neigh0_v, neigh1_v, tbuf_i, sem):
    wid = lax.axis_index("s") * NUM_CORES + lax.axis_index("c")
    base = wid * SEEDS_PER_W

    pltpu.sync_copy(node_ids_hbm.at[pl.ds(base, SEEDS_PER_W)], idx0_v)
    cp_n0 = pltpu.async_copy(neigh_hbm.at[idx0_v], neigh0_v, sem)
    cp_n0.wait()

    for g in range(GROUPS):
        _transpose16(
            lambda i: neigh0_v[g * LANES + i, :],
            lambda d, v: idx1_v.__setitem__((d, pl.ds(g * LANES, LANES)), v),
        )

    cps1 = [
        pltpu.async_copy(neigh_hbm.at[idx1_v.at[t]], neigh1_v.at[t], sem)
        for t in range(POOL_T)
    ]

    pltpu.sync_copy(idx0_v, stackT.at[0, pl.ds(HOP1 + base, SEEDS_PER_W)])
    pltpu.sync_copy(idx1_v, stackT.at[pl.ds(1, POOL_T), pl.ds(HOP1 + base, SEEDS_PER_W)])
    for t in range(POOL_T):
        pltpu.sync_copy(idx1_v.at[t],
                        stackT.at[0, pl.ds(t * N_SEEDS + base, SEEDS_PER_W)])

    for cp in cps1:
        cp.wait()

    def _chunk_out(t, carry):
        col0 = pl.multiple_of(t * N_SEEDS + base, SEEDS_PER_W)
        for g in range(GROUPS):
            _transpose16(
                lambda i: neigh1_v[t, g * LANES + i, :],
                lambda d, v: tbuf_i.__setitem__((d, pl.ds(g * LANES, LANES)), v),
            )
        pltpu.sync_copy(tbuf_i, stackT.at[pl.ds(1, POOL_T), pl.ds(col0, SEEDS_PER_W)])
        return carry

    lax.fori_loop(0, POOL_T, _chunk_out, 0)


@functools.partial(
    pl.kernel,
    out_type=(
        jax.ShapeDtypeStruct((POOL_T, TOTAL), jnp.float32),
        jax.ShapeDtypeStruct((POOL_T, N_SEEDS), jnp.float32),
    ),
    mesh=_MESH,
    compiler_params=pltpu.CompilerParams(use_tc_tiling_on_sc=False),
    scratch_types=[
        pltpu.VMEM((SEEDS_PER_W,), jnp.int32),
        pltpu.VMEM((POOL_T, SEEDS_PER_W), jnp.int32),
        pltpu.VMEM((SEEDS_PER_W, POOL_T), jnp.float32),
        pltpu.VMEM((POOL_T, SEEDS_PER_W), jnp.float32),
        pltpu.VMEM((POOL_T, SEEDS_PER_W, POOL_T), jnp.float32),
        pltpu.VMEM((POOL_T, SEEDS_PER_W), jnp.float32),
        pltpu.SemaphoreType.DMA,
    ],
)
def _alpha_sc(node_ids_hbm, alpha_hbm, stackT_hbm, alphaT, alpha0T,
              idx0_v, idx1_v, alpha0_v, a0T_v, alpha1_v, tbuf_f, sem):
    wid = lax.axis_index("s") * NUM_CORES + lax.axis_index("c")
    base = wid * SEEDS_PER_W

    pltpu.sync_copy(node_ids_hbm.at[pl.ds(base, SEEDS_PER_W)], idx0_v)
    cp_a0 = pltpu.async_copy(alpha_hbm.at[idx0_v], alpha0_v, sem)

    cps1 = []
    for t in range(POOL_T):
        pltpu.sync_copy(stackT_hbm.at[0, pl.ds(t * N_SEEDS + base, SEEDS_PER_W)],
                        idx1_v.at[t])
        cps1.append(pltpu.async_copy(alpha_hbm.at[idx1_v.at[t]], alpha1_v.at[t], sem))

    cp_a0.wait()
    for g in range(GROUPS):
        _transpose16(
            lambda i: alpha0_v[g * LANES + i, :],
            lambda d, v: a0T_v.__setitem__((d, pl.ds(g * LANES, LANES)), v),
        )
    pltpu.sync_copy(a0T_v, alphaT.at[:, pl.ds(HOP1 + base, SEEDS_PER_W)])
    pltpu.sync_copy(a0T_v, alpha0T.at[:, pl.ds(base, SEEDS_PER_W)])

    for cp in cps1:
        cp.wait()

    def _chunk_out(t, carry):
        col0 = pl.multiple_of(t * N_SEEDS + base, SEEDS_PER_W)
        for g in range(GROUPS):
            _transpose16(
                lambda i: alpha1_v[t, g * LANES + i, :],
                lambda d, v: tbuf_f.__setitem__((d, pl.ds(g * LANES, LANES)), v),
            )
        pltpu.sync_copy(tbuf_f, alphaT.at[:, pl.ds(col0, SEEDS_PER_W)])
        return carry

    lax.fori_loop(0, POOL_T, _chunk_out, 0)


def kernel(node_ids, neigh_table, alpha_table):
    stackT = _neigh_sc(node_ids, neigh_table)
    alphaT, alpha0T = _alpha_sc(node_ids, alpha_table, stackT)
    return stackT.T, alpha0T.T, alphaT.T
